# Initial kernel scaffold; baseline (speedup 1.0000x reference)
import jax
import jax.numpy as jnp
from jax.experimental import pallas as pl


def kernel(text, embed_table):
    raise NotImplementedError("write your pallas kernel here")


# speedup vs baseline: 2.8459x; 2.8459x over previous
import functools

import jax
import jax.numpy as jnp
from jax import lax
from jax.experimental import pallas as pl
from jax.experimental.pallas import tpu as pltpu
from jax.experimental.pallas import tpu_sc as plsc


def _pe_table(seq_len, dm):
    def body(o_ref):
        pos = lax.broadcasted_iota(jnp.int32, (seq_len, dm), 0).astype(jnp.float32)
        col = lax.broadcasted_iota(jnp.int32, (seq_len, dm), 1)
        expn = (col >> 1).astype(jnp.float32) * (2.0 / dm)
        base = jnp.exp(expn * jnp.log(pos * (1.0 / 10000.0)))
        base = jnp.where(pos == 0.0, jnp.where(expn == 0.0, 1.0, 0.0), base)
        o_ref[...] = jnp.where((col & 1) == 0, jnp.sin(base), jnp.cos(base))

    return pl.pallas_call(
        body, out_shape=jax.ShapeDtypeStruct((seq_len, dm), jnp.float32)
    )()


def _sc_embed(idx3, table, pe):
    nw, nb, ch = idx3.shape
    dm = table.shape[1]
    seq_len = pe.shape[0]
    n_tok = nw * nb * ch

    mesh = plsc.VectorSubcoreMesh(core_axis_name="c", subcore_axis_name="s")
    info = plsc.get_sparse_core_info()
    ncores = info.num_cores

    @functools.partial(
        pl.kernel,
        mesh=mesh,
        out_type=jax.ShapeDtypeStruct((n_tok, dm), jnp.float32),
        scratch_types=[
            pltpu.VMEM((nb, ch), jnp.int32),
            pltpu.VMEM((ch, dm), jnp.float32),
            pltpu.VMEM((2, ch, dm), jnp.float32),
            pltpu.SemaphoreType.DMA,
            pltpu.SemaphoreType.DMA,
            pltpu.SemaphoreType.DMA,
            pltpu.SemaphoreType.DMA,
        ],
    )
    def body(idx_hbm, table_hbm, pe_hbm, out_hbm, idx_v, pe_v, bufs, g0, g1, o0, o1):
        wid = lax.axis_index("s") * ncores + lax.axis_index("c")
        gsem = (g0, g1)
        osem = (o0, o1)
        pltpu.sync_copy(idx_hbm.at[wid], idx_v)
        gathers = {}
        outs = {}
        gathers[0] = pltpu.async_copy(table_hbm.at[idx_v.at[0]], bufs.at[0], g0)
        pltpu.sync_copy(pe_hbm.at[pl.ds(wid * ch, ch)], pe_v)

        def add_pe(buf):
            def row_body(r, carry):
                for g in range(dm // 16):
                    sl = pl.ds(g * 16, 16)
                    plsc.addupdate(buf.at[r, sl], pe_v[r, sl])
                return carry

            lax.fori_loop(0, ch, row_body, 0, unroll=2)

        for c in range(nb):
            s = c % 2
            if c + 1 < nb:
                if c >= 1:
                    outs[c - 1].wait()
                gathers[c + 1] = pltpu.async_copy(
                    table_hbm.at[idx_v.at[c + 1]], bufs.at[(c + 1) % 2], gsem[(c + 1) % 2]
                )
            gathers[c].wait()
            add_pe(bufs.at[s])
            outs[c] = pltpu.async_copy(
                bufs.at[s], out_hbm.at[pl.ds(c * seq_len + wid * ch, ch)], osem[s]
            )
        outs[nb - 2].wait()
        outs[nb - 1].wait()

    return body(idx3, table, pe)


def kernel(text, embed_table):
    b, seq_len = text.shape
    dm = embed_table.shape[1]
    pe = _pe_table(seq_len, dm)
    info = plsc.get_sparse_core_info()
    nw = info.num_cores * info.num_subcores
    ch = seq_len // nw
    idx3 = text.reshape(b, nw, ch).transpose(1, 0, 2)
    out = _sc_embed(idx3, embed_table, pe)
    return out.reshape(b, seq_len, dm)


# speedup vs baseline: 3.7655x; 1.3231x over previous
import functools

import numpy as np
import jax
import jax.numpy as jnp
from jax import lax
from jax.experimental import pallas as pl
from jax.experimental.pallas import tpu as pltpu
from jax.experimental.pallas import tpu_sc as plsc


def _pe_table(seq_len, dm):
    pos = np.arange(seq_len, dtype=np.float32)[:, None]
    col = np.arange(dm)[None, :]
    expn = ((col // 2).astype(np.float32) * (2.0 / dm)).astype(np.float32)
    base = np.power(pos / 10000.0, expn, dtype=np.float32)
    pe = np.where(col % 2 == 0, np.sin(base), np.cos(base)).astype(np.float32)
    return jnp.asarray(pe)


def _sc_embed(text, table, pe):
    nb, seq_len = text.shape
    dm = table.shape[1]
    n_tok = nb * seq_len

    mesh = plsc.VectorSubcoreMesh(core_axis_name="c", subcore_axis_name="s")
    info = plsc.get_sparse_core_info()
    ncores = info.num_cores
    nw = info.num_cores * info.num_subcores
    ch = seq_len // nw

    @functools.partial(
        pl.kernel,
        mesh=mesh,
        out_type=jax.ShapeDtypeStruct((n_tok, dm), jnp.float32),
        scratch_types=[
            pltpu.VMEM((nb, ch), jnp.int32),
            pltpu.VMEM((ch, dm), jnp.float32),
            pltpu.VMEM((2, ch, dm), jnp.float32),
            pltpu.SemaphoreType.DMA,
            pltpu.SemaphoreType.DMA,
            pltpu.SemaphoreType.DMA,
            pltpu.SemaphoreType.DMA,
        ],
    )
    def body(idx_hbm, table_hbm, pe_hbm, out_hbm, idx_v, pe_v, bufs, g0, g1, o0, o1):
        wid = lax.axis_index("s") * ncores + lax.axis_index("c")
        gsem = (g0, g1)
        osem = (o0, o1)
        pltpu.sync_copy(idx_hbm.at[0, pl.ds(wid * ch, ch)], idx_v.at[0])
        gathers = {}
        outs = {}
        gathers[0] = pltpu.async_copy(table_hbm.at[idx_v.at[0]], bufs.at[0], g0)
        for b in range(1, nb):
            pltpu.sync_copy(idx_hbm.at[b, pl.ds(wid * ch, ch)], idx_v.at[b])
        pltpu.sync_copy(pe_hbm.at[pl.ds(wid * ch, ch)], pe_v)

        def add_pe(buf):
            def row_body(r, carry):
                for g in range(dm // 16):
                    sl = pl.ds(g * 16, 16)
                    plsc.addupdate(buf.at[r, sl], pe_v[r, sl])
                return carry

            lax.fori_loop(0, ch, row_body, 0, unroll=2)

        for c in range(nb):
            s = c % 2
            if c + 1 < nb:
                if c >= 1:
                    outs[c - 1].wait()
                gathers[c + 1] = pltpu.async_copy(
                    table_hbm.at[idx_v.at[c + 1]], bufs.at[(c + 1) % 2], gsem[(c + 1) % 2]
                )
            gathers[c].wait()
            add_pe(bufs.at[s])
            outs[c] = pltpu.async_copy(
                bufs.at[s], out_hbm.at[pl.ds(c * seq_len + wid * ch, ch)], osem[s]
            )
        outs[nb - 2].wait()
        outs[nb - 1].wait()

    return body(text, table, pe)


def kernel(text, embed_table):
    b, seq_len = text.shape
    dm = embed_table.shape[1]
    pe = _pe_table(seq_len, dm)
    out = _sc_embed(text, embed_table, pe)
    return out.reshape(b, seq_len, dm)


# speedup vs baseline: 3.8891x; 1.0328x over previous
import functools

import numpy as np
import jax
import jax.numpy as jnp
from jax import lax
from jax.experimental import pallas as pl
from jax.experimental.pallas import tpu as pltpu
from jax.experimental.pallas import tpu_sc as plsc


def _pe_table(seq_len, dm):
    pos = np.arange(seq_len, dtype=np.float32)[:, None]
    col = np.arange(dm)[None, :]
    expn = ((col // 2).astype(np.float32) * (2.0 / dm)).astype(np.float32)
    base = np.power(pos / 10000.0, expn, dtype=np.float32)
    pe = np.where(col % 2 == 0, np.sin(base), np.cos(base)).astype(np.float32)
    return jnp.asarray(pe)


def _sc_embed(text, table, pe):
    nb, seq_len = text.shape
    dm = table.shape[1]
    n_tok = nb * seq_len

    mesh = plsc.VectorSubcoreMesh(core_axis_name="c", subcore_axis_name="s")
    info = plsc.get_sparse_core_info()
    ncores = info.num_cores
    nw = info.num_cores * info.num_subcores
    ch = seq_len // nw

    @functools.partial(
        pl.kernel,
        mesh=mesh,
        out_type=jax.ShapeDtypeStruct((n_tok, dm), jnp.float32),
        scratch_types=[
            pltpu.VMEM((nb, ch), jnp.int32),
            pltpu.VMEM((ch, dm), jnp.float32),
            pltpu.VMEM((2, ch, dm), jnp.float32),
            pltpu.SemaphoreType.DMA,
            pltpu.SemaphoreType.DMA,
            pltpu.SemaphoreType.DMA,
            pltpu.SemaphoreType.DMA,
        ],
    )
    def body(idx_hbm, table_hbm, pe_hbm, out_hbm, idx_v, pe_v, bufs, g0, g1, o0, o1):
        wid = lax.axis_index("s") * ncores + lax.axis_index("c")
        gsem = (g0, g1)
        osem = (o0, o1)
        pltpu.sync_copy(idx_hbm.at[0, pl.ds(wid * ch, ch)], idx_v.at[0])
        gathers = {}
        outs = {}
        gathers[0] = pltpu.async_copy(table_hbm.at[idx_v.at[0]], bufs.at[0], g0)
        for b in range(1, nb):
            pltpu.sync_copy(idx_hbm.at[b, pl.ds(wid * ch, ch)], idx_v.at[b])
        pltpu.sync_copy(pe_hbm.at[pl.ds(wid * ch, ch)], pe_v)

        def add_pe(buf):
            @plsc.parallel_loop(0, ch, 1, unroll=4)
            def _row(r):
                for g in range(dm // 16):
                    sl = pl.ds(g * 16, 16)
                    plsc.addupdate(buf.at[r, sl], pe_v[r, sl])

        for c in range(nb):
            s = c % 2
            if c + 1 < nb:
                if c >= 1:
                    outs[c - 1].wait()
                gathers[c + 1] = pltpu.async_copy(
                    table_hbm.at[idx_v.at[c + 1]], bufs.at[(c + 1) % 2], gsem[(c + 1) % 2]
                )
            gathers[c].wait()
            add_pe(bufs.at[s])
            outs[c] = pltpu.async_copy(
                bufs.at[s], out_hbm.at[pl.ds(c * seq_len + wid * ch, ch)], osem[s]
            )
        outs[nb - 2].wait()
        outs[nb - 1].wait()

    return body(text, table, pe)


def kernel(text, embed_table):
    b, seq_len = text.shape
    dm = embed_table.shape[1]
    pe = _pe_table(seq_len, dm)
    out = _sc_embed(text, embed_table, pe)
    return out.reshape(b, seq_len, dm)
